# baseline (speedup 1.0000x reference)
import jax
import jax.numpy as jnp
from jax import lax
from jax.experimental import pallas as pl
from jax.experimental.pallas import tpu as pltpu

_B, _C, _HW = 64, 768, 196
_A = 16
_CH = 7
_NCH = _HW // _CH
_NBUF = 4


def _head_body(xt_ref, wt_ref, g_ref, act_ref, sel_ref,
               b0, b1, b2, b3, acc_ref, sems):
    bufs = (b0, b1, b2, b3)

    def copy(c):
        return pltpu.make_async_copy(
            xt_ref.at[pl.ds(c * _CH, _CH)], bufs[c % _NBUF],
            sems.at[c % _NBUF])

    for c in range(_NBUF):
        copy(c).start()
    for c in range(_NCH):
        copy(c).wait()
        part = jnp.sum(jnp.maximum(bufs[c % _NBUF][...], 0.0), axis=0)
        if c == 0:
            acc_ref[...] = part
        else:
            acc_ref[...] += part
        if c + _NBUF < _NCH:
            copy(c + _NBUF).start()

    pooled = acc_ref[...] * (1.0 / _HW)
    logits = lax.dot_general(
        pooled, wt_ref[...], (((1,), (0,)), ((), ())),
        preferred_element_type=jnp.float32,
        precision=lax.Precision.HIGHEST)
    m = jnp.max(logits, axis=1, keepdims=True)
    e = jnp.exp(logits - m)
    p = e / jnp.sum(e, axis=1, keepdims=True)
    idx = lax.broadcasted_iota(jnp.int32, p.shape, 1)
    cand = jnp.where(p >= jnp.max(p, axis=1, keepdims=True), idx, _A)
    act = jnp.min(cand, axis=1, keepdims=True)
    act_ref[...] = act
    g = g_ref[...]
    sel = jnp.broadcast_to(g[0][None, :], (_B, _C))
    for a in range(1, _A):
        sel = jnp.where(act == a, g[a][None, :], sel)
    sel_ref[...] = sel


def kernel(x, fc1_weight, channel_gates):
    xt = jnp.transpose(x, (2, 3, 0, 1)).reshape(_HW, _B, _C)
    wt = fc1_weight.T
    actions2d, selected = pl.pallas_call(
        _head_body,
        in_specs=[
            pl.BlockSpec(memory_space=pltpu.HBM),
            pl.BlockSpec(memory_space=pltpu.VMEM),
            pl.BlockSpec(memory_space=pltpu.VMEM),
        ],
        out_specs=[
            pl.BlockSpec(memory_space=pltpu.VMEM),
            pl.BlockSpec(memory_space=pltpu.VMEM),
        ],
        out_shape=[
            jax.ShapeDtypeStruct((_B, 1), jnp.int32),
            jax.ShapeDtypeStruct((_B, _C), jnp.float32),
        ],
        scratch_shapes=(
            [pltpu.VMEM((_CH, _B, _C), jnp.float32) for _ in range(_NBUF)]
            + [pltpu.VMEM((_B, _C), jnp.float32),
               pltpu.SemaphoreType.DMA((_NBUF,))]
        ),
    )(xt, wt, channel_gates)
    return actions2d.reshape(_B), selected


# speedup vs baseline: 1.0206x; 1.0206x over previous
import jax
import jax.numpy as jnp
from jax import lax
from jax.experimental import pallas as pl
from jax.experimental.pallas import tpu as pltpu

_B, _C, _HW = 64, 768, 196
_A = 16
_K = 49
_S = _HW // _K


def _head_body(x_ref, wt_ref, g_ref, act_ref, sel_ref, acc_ref):
    i = pl.program_id(0)
    part = jnp.sum(jnp.maximum(x_ref[...], 0.0), axis=0)

    @pl.when(i == 0)
    def _():
        acc_ref[...] = part

    @pl.when(i > 0)
    def _():
        acc_ref[...] += part

    @pl.when(i == _S - 1)
    def _():
        pooled = acc_ref[...] * (1.0 / _HW)
        logits = lax.dot_general(
            pooled, wt_ref[...], (((1,), (0,)), ((), ())),
            preferred_element_type=jnp.float32,
            precision=lax.Precision.HIGHEST)
        m = jnp.max(logits, axis=1, keepdims=True)
        e = jnp.exp(logits - m)
        p = e / jnp.sum(e, axis=1, keepdims=True)
        idx = lax.broadcasted_iota(jnp.int32, p.shape, 1)
        cand = jnp.where(p >= jnp.max(p, axis=1, keepdims=True), idx, _A)
        act = jnp.min(cand, axis=1, keepdims=True)
        act_ref[...] = act
        g = g_ref[...]
        sel = jnp.broadcast_to(g[0][None, :], (_B, _C))
        for a in range(1, _A):
            sel = jnp.where(act == a, g[a][None, :], sel)
        sel_ref[...] = sel


def kernel(x, fc1_weight, channel_gates):
    xt = jnp.transpose(x, (2, 3, 0, 1)).reshape(_HW, _B, _C)
    wt = fc1_weight.T
    actions2d, selected = pl.pallas_call(
        _head_body,
        grid=(_S,),
        in_specs=[
            pl.BlockSpec((_K, _B, _C), lambda i: (i, 0, 0)),
            pl.BlockSpec((_C, _A), lambda i: (0, 0)),
            pl.BlockSpec((_A, _C), lambda i: (0, 0)),
        ],
        out_specs=[
            pl.BlockSpec((_B, 1), lambda i: (0, 0)),
            pl.BlockSpec((_B, _C), lambda i: (0, 0)),
        ],
        out_shape=[
            jax.ShapeDtypeStruct((_B, 1), jnp.int32),
            jax.ShapeDtypeStruct((_B, _C), jnp.float32),
        ],
        scratch_shapes=[pltpu.VMEM((_B, _C), jnp.float32)],
    )(xt, wt, channel_gates)
    return actions2d.reshape(_B), selected
